# Initial kernel scaffold; baseline (speedup 1.0000x reference)
import jax
import jax.numpy as jnp
from jax.experimental import pallas as pl


def kernel(x, edge_index, edge_attr, We, be, W1, b1, W2, b2, gamma, beta):
    raise NotImplementedError("write your pallas kernel here")


# speedup vs baseline: 1.6747x; 1.6747x over previous
import functools

import jax
import jax.numpy as jnp
from jax import lax
from jax.experimental import pallas as pl
from jax.experimental.pallas import tpu as pltpu
from jax.experimental.pallas import tpu_sc as plsc

N = 10000
E = 320000
D = 128
DE = 16

NC = 2
NS = 16
L = 16
NW = NC * NS
DPT = D // NW
C = 1280
NCHUNK = E // C


def _edge_proj_body(ea_ref, we_ref, be_ref, out_ref):
    e = jnp.dot(ea_ref[...], we_ref[...], preferred_element_type=jnp.float32)
    out_ref[...] = (e + be_ref[...]).T


def _edge_proj(edge_attr, We, be2d):
    ce = 2560
    grid = E // ce
    return pl.pallas_call(
        _edge_proj_body,
        grid=(grid,),
        in_specs=[
            pl.BlockSpec((ce, DE), lambda i: (i, 0)),
            pl.BlockSpec((DE, D), lambda i: (0, 0)),
            pl.BlockSpec((1, D), lambda i: (0, 0)),
        ],
        out_specs=pl.BlockSpec((D, ce), lambda i: (0, i)),
        out_shape=jax.ShapeDtypeStruct((D, E), jnp.float32),
    )(edge_attr, We, be2d)


def _xpose_body(x_ref, out_ref):
    out_ref[...] = x_ref[...].T


def _xpose(x):
    return pl.pallas_call(
        _xpose_body,
        grid=(1,),
        in_specs=[pl.BlockSpec((N, D), lambda i: (0, 0))],
        out_specs=pl.BlockSpec((D, N), lambda i: (0, 0)),
        out_shape=jax.ShapeDtypeStruct((D, N), jnp.float32),
    )(x)


def _sc_body(xT_hbm, ei_hbm, eT_hbm, out_hbm, xv, av, ev, sv, es0, es1,
             is0, is1, xsem):
    wid = lax.axis_index("s") * NC + lax.axis_index("c")
    esems = (es0, es1)
    isems = (is0, is1)

    xcp = pltpu.make_async_copy(xT_hbm.at[wid], xv, xsem)
    xcp.start()

    zeros = jnp.zeros((L,), jnp.float32)

    def zb(i, carry):
        for d in range(DPT):
            av[d, pl.ds(i * L, L)] = zeros
        return carry

    lax.fori_loop(0, N // L, zb, 0)
    xcp.wait()

    def e_copy(ci, b):
        return pltpu.make_async_copy(
            eT_hbm.at[wid, :, pl.ds(ci * C, C)], ev.at[b], esems[b])

    def i_copy(ci, b):
        return pltpu.make_async_copy(
            ei_hbm.at[:, pl.ds(ci * C, C)], sv.at[b], isems[b])

    for b in range(2):
        e_copy(b, b).start()
        i_copy(b, b).start()

    dfull = [jnp.full((L,), d, jnp.int32) for d in range(DPT)]

    def outer(j, carry):
        for b in range(2):
            ci = 2 * j + b
            e_copy(ci, b).wait()
            i_copy(ci, b).wait()

            def gb(g, c2):
                off = g * L
                s = sv[b, 0, pl.ds(off, L)]
                t = sv[b, 1, pl.ds(off, L)]
                for d in range(DPT):
                    evd = ev[b, d, pl.ds(off, L)]
                    xg = plsc.load_gather(xv, [dfull[d], s])
                    m = jnp.maximum(xg + evd, 0.0)
                    plsc.addupdate_scatter(av, [dfull[d], t], m)
                return c2

            lax.fori_loop(0, C // L, gb, 0)

            nci = ci + 2

            @pl.when(nci < NCHUNK)
            def _():
                e_copy(nci, b).start()
                i_copy(nci, b).start()
        return carry

    lax.fori_loop(0, NCHUNK // 2, outer, 0)
    pltpu.sync_copy(av, out_hbm.at[wid])


def _sc_aggr(xT, ei, eT):
    mesh = plsc.VectorSubcoreMesh(core_axis_name="c", subcore_axis_name="s")
    f = functools.partial(
        pl.kernel,
        out_type=jax.ShapeDtypeStruct((NW, DPT, N), jnp.float32),
        mesh=mesh,
        compiler_params=pltpu.CompilerParams(needs_layout_passes=False),
        scratch_types=[
            pltpu.VMEM((DPT, N), jnp.float32),
            pltpu.VMEM((DPT, N), jnp.float32),
            pltpu.VMEM((2, DPT, C), jnp.float32),
            pltpu.VMEM((2, 2, C), jnp.int32),
            pltpu.SemaphoreType.DMA,
            pltpu.SemaphoreType.DMA,
            pltpu.SemaphoreType.DMA,
            pltpu.SemaphoreType.DMA,
            pltpu.SemaphoreType.DMA,
        ],
    )(_sc_body)
    return f(xT.reshape(NW, DPT, N), ei, eT.reshape(NW, DPT, E))


def _mlp_body(x_ref, at_ref, w1_ref, b1_ref, w2_ref, b2_ref, g_ref, bt_ref,
              o_ref):
    x = x_ref[...]
    h = x + at_ref[...].T
    h1 = jnp.maximum(
        jnp.dot(h, w1_ref[...], preferred_element_type=jnp.float32)
        + b1_ref[...], 0.0)
    h2 = jnp.dot(h1, w2_ref[...], preferred_element_type=jnp.float32) \
        + b2_ref[...]
    y = jnp.maximum(h2, 0.0) + x
    mu = jnp.mean(y, axis=-1, keepdims=True)
    yc = y - mu
    var = jnp.mean(yc * yc, axis=-1, keepdims=True)
    o_ref[...] = yc * lax.rsqrt(var + 1e-5) * g_ref[...] + bt_ref[...]


def _mlp(x, aggrT, W1, b1, W2, b2, gamma, beta):
    bn = 1024
    grid = pl.cdiv(N, bn)
    return pl.pallas_call(
        _mlp_body,
        grid=(grid,),
        in_specs=[
            pl.BlockSpec((bn, D), lambda i: (i, 0)),
            pl.BlockSpec((D, bn), lambda i: (0, i)),
            pl.BlockSpec((D, D), lambda i: (0, 0)),
            pl.BlockSpec((1, D), lambda i: (0, 0)),
            pl.BlockSpec((D, D), lambda i: (0, 0)),
            pl.BlockSpec((1, D), lambda i: (0, 0)),
            pl.BlockSpec((1, D), lambda i: (0, 0)),
            pl.BlockSpec((1, D), lambda i: (0, 0)),
        ],
        out_specs=pl.BlockSpec((bn, D), lambda i: (i, 0)),
        out_shape=jax.ShapeDtypeStruct((N, D), jnp.float32),
    )(x, aggrT, W1, b1, W2, b2, gamma, beta)


def kernel(x, edge_index, edge_attr, We, be, W1, b1, W2, b2, gamma, beta):
    ei = edge_index.astype(jnp.int32)
    eT = _edge_proj(edge_attr, We, be.reshape(1, D))
    xT = _xpose(x)
    aggrT = _sc_aggr(xT, ei, eT)
    return _mlp(x, aggrT.reshape(D, N), W1, b1.reshape(1, D), W2,
                b2.reshape(1, D), gamma.reshape(1, D), beta.reshape(1, D))


# speedup vs baseline: 3.5188x; 2.1012x over previous
import functools

import jax
import jax.numpy as jnp
from jax import lax
from jax.experimental import pallas as pl
from jax.experimental.pallas import tpu as pltpu
from jax.experimental.pallas import tpu_sc as plsc

N = 10000
E = 320000
D = 128
DE = 16

NC = 2
NS = 16
L = 16
NW = NC * NS
DPT = D // NW
C = 1280
NCHUNK = E // C


def _edge_proj_body(ea_ref, we_ref, be_ref, out_ref):
    e = jnp.dot(ea_ref[...], we_ref[...], preferred_element_type=jnp.float32)
    out_ref[...] = (e + be_ref[...]).T


def _edge_proj(edge_attr, We, be2d):
    ce = 2560
    grid = E // ce
    return pl.pallas_call(
        _edge_proj_body,
        grid=(grid,),
        in_specs=[
            pl.BlockSpec((ce, DE), lambda i: (i, 0)),
            pl.BlockSpec((DE, D), lambda i: (0, 0)),
            pl.BlockSpec((1, D), lambda i: (0, 0)),
        ],
        out_specs=pl.BlockSpec((D, ce), lambda i: (0, i)),
        out_shape=jax.ShapeDtypeStruct((D, E), jnp.float32),
    )(edge_attr, We, be2d)


def _xpose_body(x_ref, out_ref):
    out_ref[...] = x_ref[...].T


def _xpose(x):
    return pl.pallas_call(
        _xpose_body,
        grid=(1,),
        in_specs=[pl.BlockSpec((N, D), lambda i: (0, 0))],
        out_specs=pl.BlockSpec((D, N), lambda i: (0, 0)),
        out_shape=jax.ShapeDtypeStruct((D, N), jnp.float32),
    )(x)


def _sc_body(xT_hbm, ei_hbm, eT_hbm, out_hbm, xv, av, ev, sv, es0, es1,
             is0, is1, xsem):
    half = lax.axis_index("c")
    pid = lax.axis_index("s")
    wid = pid * NC + half
    esems = (es0, es1)
    isems = (is0, is1)

    xcp = pltpu.make_async_copy(xT_hbm.at[wid], xv, xsem)
    xcp.start()

    def e_copy(ci, b):
        return pltpu.make_async_copy(
            eT_hbm.at[pid, :, pl.ds(ci * C, C)], ev.at[b], esems[b])

    def i_copy(ci, b):
        return pltpu.make_async_copy(
            ei_hbm.at[:, pl.ds(ci * C, C)], sv.at[b], isems[b])

    for b in range(2):
        e_copy(b, b).start()
        i_copy(b, b).start()

    zeros = jnp.zeros((L,), jnp.float32)

    @plsc.parallel_loop(0, N // L, unroll=8)
    def _zb(i):
        for d in range(DPT):
            av[d, pl.ds(i * L, L)] = zeros

    xcp.wait()

    rbase = DPT * half
    dfull = [jnp.full((L,), d, jnp.int32) for d in range(DPT)]

    def outer(j, carry):
        for b in range(2):
            ci = 2 * j + b
            e_copy(ci, b).wait()
            i_copy(ci, b).wait()

            @plsc.parallel_loop(0, C // L, unroll=4)
            def _gb(g):
                off = g * L
                s = sv[b, 0, pl.ds(off, L)]
                t = sv[b, 1, pl.ds(off, L)]
                for d in range(DPT):
                    evd = ev[b, rbase + d, pl.ds(off, L)]
                    xg = plsc.load_gather(xv, [dfull[d], s])
                    m = jnp.maximum(xg + evd, 0.0)
                    plsc.addupdate_scatter(av, [dfull[d], t], m)

            nci = ci + 2

            @pl.when(nci < NCHUNK)
            def _():
                e_copy(nci, b).start()
                i_copy(nci, b).start()
        return carry

    lax.fori_loop(0, NCHUNK // 2, outer, 0)
    pltpu.sync_copy(av, out_hbm.at[wid])


def _sc_aggr(xT, ei, eT):
    mesh = plsc.VectorSubcoreMesh(core_axis_name="c", subcore_axis_name="s")
    f = functools.partial(
        pl.kernel,
        out_type=jax.ShapeDtypeStruct((NW, DPT, N), jnp.float32),
        mesh=mesh,
        compiler_params=pltpu.CompilerParams(needs_layout_passes=False),
        scratch_types=[
            pltpu.VMEM((DPT, N), jnp.float32),
            pltpu.VMEM((DPT, N), jnp.float32),
            pltpu.VMEM((2, 2 * DPT, C), jnp.float32),
            pltpu.VMEM((2, 2, C), jnp.int32),
            pltpu.SemaphoreType.DMA,
            pltpu.SemaphoreType.DMA,
            pltpu.SemaphoreType.DMA,
            pltpu.SemaphoreType.DMA,
            pltpu.SemaphoreType.DMA,
        ],
    )(_sc_body)
    return f(xT.reshape(NW, DPT, N), ei, eT.reshape(NS, NC * DPT, E))


def _mlp_body(x_ref, at_ref, w1_ref, b1_ref, w2_ref, b2_ref, g_ref, bt_ref,
              o_ref):
    x = x_ref[...]
    h = x + at_ref[...].T
    h1 = jnp.maximum(
        jnp.dot(h, w1_ref[...], preferred_element_type=jnp.float32)
        + b1_ref[...], 0.0)
    h2 = jnp.dot(h1, w2_ref[...], preferred_element_type=jnp.float32) \
        + b2_ref[...]
    y = jnp.maximum(h2, 0.0) + x
    mu = jnp.mean(y, axis=-1, keepdims=True)
    yc = y - mu
    var = jnp.mean(yc * yc, axis=-1, keepdims=True)
    o_ref[...] = yc * lax.rsqrt(var + 1e-5) * g_ref[...] + bt_ref[...]


def _mlp(x, aggrT, W1, b1, W2, b2, gamma, beta):
    bn = 1024
    grid = pl.cdiv(N, bn)
    return pl.pallas_call(
        _mlp_body,
        grid=(grid,),
        in_specs=[
            pl.BlockSpec((bn, D), lambda i: (i, 0)),
            pl.BlockSpec((D, bn), lambda i: (0, i)),
            pl.BlockSpec((D, D), lambda i: (0, 0)),
            pl.BlockSpec((1, D), lambda i: (0, 0)),
            pl.BlockSpec((D, D), lambda i: (0, 0)),
            pl.BlockSpec((1, D), lambda i: (0, 0)),
            pl.BlockSpec((1, D), lambda i: (0, 0)),
            pl.BlockSpec((1, D), lambda i: (0, 0)),
        ],
        out_specs=pl.BlockSpec((bn, D), lambda i: (i, 0)),
        out_shape=jax.ShapeDtypeStruct((N, D), jnp.float32),
    )(x, aggrT, W1, b1, W2, b2, gamma, beta)


def kernel(x, edge_index, edge_attr, We, be, W1, b1, W2, b2, gamma, beta):
    ei = edge_index.astype(jnp.int32)
    eT = _edge_proj(edge_attr, We, be.reshape(1, D))
    xT = _xpose(x)
    aggrT = _sc_aggr(xT, ei, eT)
    return _mlp(x, aggrT.reshape(D, N), W1, b1.reshape(1, D), W2,
                b2.reshape(1, D), gamma.reshape(1, D), beta.reshape(1, D))


# speedup vs baseline: 4.5533x; 1.2940x over previous
import functools

import jax
import jax.numpy as jnp
from jax import lax
from jax.experimental import pallas as pl
from jax.experimental.pallas import tpu as pltpu
from jax.experimental.pallas import tpu_sc as plsc

N = 10000
E = 320000
D = 128
DE = 16

NC = 2
NS = 16
L = 16
NW = NC * NS
DPT = D // NW
C = 1280
NCHUNK = E // C


def _edge_proj_body(weT_ref, eaT_ref, be_ref, out_ref):
    out_ref[...] = lax.dot_general(
        weT_ref[...], eaT_ref[...], (((1,), (0,)), ((), ())),
        preferred_element_type=jnp.float32) + be_ref[...]


def _edge_proj(eaT, WeT, be2d):
    ce = 2560
    grid = E // ce
    return pl.pallas_call(
        _edge_proj_body,
        grid=(grid,),
        in_specs=[
            pl.BlockSpec((D, DE), lambda i: (0, 0)),
            pl.BlockSpec((DE, ce), lambda i: (0, i)),
            pl.BlockSpec((D, 1), lambda i: (0, 0)),
        ],
        out_specs=pl.BlockSpec((D, ce), lambda i: (0, i)),
        out_shape=jax.ShapeDtypeStruct((D, E), jnp.float32),
    )(WeT, eaT, be2d)


def _xpose_body(x_ref, out_ref):
    out_ref[...] = x_ref[...].T


def _xpose(x):
    return pl.pallas_call(
        _xpose_body,
        grid=(1,),
        in_specs=[pl.BlockSpec((N, D), lambda i: (0, 0))],
        out_specs=pl.BlockSpec((D, N), lambda i: (0, 0)),
        out_shape=jax.ShapeDtypeStruct((D, N), jnp.float32),
    )(x)


def _sc_body(xT_hbm, ei_hbm, eT_hbm, out_hbm, xv, av, ev, sv, es0, es1,
             is0, is1, xsem):
    half = lax.axis_index("c")
    pid = lax.axis_index("s")
    wid = pid * NC + half
    esems = (es0, es1)
    isems = (is0, is1)

    xcp = pltpu.make_async_copy(xT_hbm.at[wid], xv, xsem)
    xcp.start()

    def e_copy(ci, b):
        return pltpu.make_async_copy(
            eT_hbm.at[pid, :, pl.ds(ci * C, C)], ev.at[b], esems[b])

    def i_copy(ci, b):
        return pltpu.make_async_copy(
            ei_hbm.at[:, pl.ds(ci * C, C)], sv.at[b], isems[b])

    for b in range(2):
        e_copy(b, b).start()
        i_copy(b, b).start()

    zeros = jnp.zeros((L,), jnp.float32)

    @plsc.parallel_loop(0, N // L, unroll=8)
    def _zb(i):
        for d in range(DPT):
            av[d, pl.ds(i * L, L)] = zeros

    xcp.wait()

    rbase = DPT * half
    dfull = [jnp.full((L,), d, jnp.int32) for d in range(DPT)]

    def outer(j, carry):
        for b in range(2):
            ci = 2 * j + b
            e_copy(ci, b).wait()
            i_copy(ci, b).wait()

            @plsc.parallel_loop(0, C // L, unroll=4)
            def _gb(g):
                off = g * L
                s = sv[b, 0, pl.ds(off, L)]
                t = sv[b, 1, pl.ds(off, L)]
                for d in range(DPT):
                    evd = ev[b, rbase + d, pl.ds(off, L)]
                    xg = plsc.load_gather(xv, [dfull[d], s])
                    m = jnp.maximum(xg + evd, 0.0)
                    plsc.addupdate_scatter(av, [dfull[d], t], m)

            nci = ci + 2

            @pl.when(nci < NCHUNK)
            def _():
                e_copy(nci, b).start()
                i_copy(nci, b).start()
        return carry

    lax.fori_loop(0, NCHUNK // 2, outer, 0)
    pltpu.sync_copy(av, out_hbm.at[wid])


def _sc_aggr(xT, ei, eT):
    mesh = plsc.VectorSubcoreMesh(core_axis_name="c", subcore_axis_name="s")
    f = functools.partial(
        pl.kernel,
        out_type=jax.ShapeDtypeStruct((NW, DPT, N), jnp.float32),
        mesh=mesh,
        compiler_params=pltpu.CompilerParams(needs_layout_passes=False),
        scratch_types=[
            pltpu.VMEM((DPT, N), jnp.float32),
            pltpu.VMEM((DPT, N), jnp.float32),
            pltpu.VMEM((2, 2 * DPT, C), jnp.float32),
            pltpu.VMEM((2, 2, C), jnp.int32),
            pltpu.SemaphoreType.DMA,
            pltpu.SemaphoreType.DMA,
            pltpu.SemaphoreType.DMA,
            pltpu.SemaphoreType.DMA,
            pltpu.SemaphoreType.DMA,
        ],
    )(_sc_body)
    return f(xT.reshape(NW, DPT, N), ei, eT.reshape(NS, NC * DPT, E))


def _mlp_body(x_ref, at_ref, w1_ref, b1_ref, w2_ref, b2_ref, g_ref, bt_ref,
              o_ref):
    x = x_ref[...]
    h = x + at_ref[...].T
    h1 = jnp.maximum(
        jnp.dot(h, w1_ref[...], preferred_element_type=jnp.float32)
        + b1_ref[...], 0.0)
    h2 = jnp.dot(h1, w2_ref[...], preferred_element_type=jnp.float32) \
        + b2_ref[...]
    y = jnp.maximum(h2, 0.0) + x
    mu = jnp.mean(y, axis=-1, keepdims=True)
    yc = y - mu
    var = jnp.mean(yc * yc, axis=-1, keepdims=True)
    o_ref[...] = yc * lax.rsqrt(var + 1e-5) * g_ref[...] + bt_ref[...]


def _mlp(x, aggrT, W1, b1, W2, b2, gamma, beta):
    bn = 1024
    grid = pl.cdiv(N, bn)
    return pl.pallas_call(
        _mlp_body,
        grid=(grid,),
        in_specs=[
            pl.BlockSpec((bn, D), lambda i: (i, 0)),
            pl.BlockSpec((D, bn), lambda i: (0, i)),
            pl.BlockSpec((D, D), lambda i: (0, 0)),
            pl.BlockSpec((1, D), lambda i: (0, 0)),
            pl.BlockSpec((D, D), lambda i: (0, 0)),
            pl.BlockSpec((1, D), lambda i: (0, 0)),
            pl.BlockSpec((1, D), lambda i: (0, 0)),
            pl.BlockSpec((1, D), lambda i: (0, 0)),
        ],
        out_specs=pl.BlockSpec((bn, D), lambda i: (i, 0)),
        out_shape=jax.ShapeDtypeStruct((N, D), jnp.float32),
    )(x, aggrT, W1, b1, W2, b2, gamma, beta)


def kernel(x, edge_index, edge_attr, We, be, W1, b1, W2, b2, gamma, beta):
    ei = edge_index.astype(jnp.int32)
    eT = _edge_proj(edge_attr.T, We.T, be.reshape(D, 1))
    xT = _xpose(x)
    aggrT = _sc_aggr(xT, ei, eT)
    return _mlp(x, aggrT.reshape(D, N), W1, b1.reshape(1, D), W2,
                b2.reshape(1, D), gamma.reshape(1, D), beta.reshape(1, D))
